# speedup vs baseline: 12.1444x; 12.1444x over previous
import jax
import jax.numpy as jnp
from jax.experimental import pallas as pl
from jax.experimental.pallas import tpu as pltpu

N = 100000
D = 128
G = 256
BLOCK = 2000
NB = N // BLOCK


def _body(x_ref, ids_ref, w_ref, b_ref, q_ref, out_ref, m_ref, d_ref, s_ref):
    i = pl.program_id(0)

    @pl.when(i == 0)
    def _init():
        m_ref[...] = jnp.full((1, G), -1e30, jnp.float32)
        d_ref[...] = jnp.zeros((1, G), jnp.float32)
        s_ref[...] = jnp.zeros((D, G), jnp.float32)

    xb = x_ref[...]
    g = jnp.tanh(
        jax.lax.dot_general(
            xb, w_ref[...], (((1,), (1,)), ((), ())),
            preferred_element_type=jnp.float32,
        )
        + b_ref[...]
    )
    score = jax.lax.dot_general(
        g, q_ref[...], (((1,), (0,)), ((), ())),
        preferred_element_type=jnp.float32,
    )

    ids = ids_ref[0]
    one_hot = ids == jax.lax.broadcasted_iota(jnp.int32, (BLOCK, G), 1)

    masked = jnp.where(one_hot, jnp.broadcast_to(score, (BLOCK, G)), -1e30)
    bm = jnp.max(masked, axis=0, keepdims=True)
    m_old = m_ref[...]
    m_new = jnp.maximum(m_old, bm)
    scale = jnp.exp(m_old - m_new)

    m_row = jnp.sum(jnp.where(one_hot, m_new, 0.0), axis=1, keepdims=True)
    e = jnp.exp(score - m_row)
    p = jnp.where(one_hot, jnp.broadcast_to(e, (BLOCK, G)), 0.0)

    d_ref[...] = d_ref[...] * scale + jnp.sum(p, axis=0, keepdims=True)
    s_ref[...] = s_ref[...] * scale + jax.lax.dot_general(
        xb, p, (((0,), (0,)), ((), ())), preferred_element_type=jnp.float32
    )
    m_ref[...] = m_new

    @pl.when(i == NB - 1)
    def _fini():
        d = d_ref[...]
        d = jnp.where(d == 0.0, 1.0, d)
        out_ref[...] = (s_ref[...] / d).T


@jax.jit
def kernel(x, graph_ptr, W, b, query):
    ids = graph_ptr.reshape(NB, BLOCK, 1)
    b2 = b.reshape(1, D)
    q2 = query.reshape(D, 1)
    return pl.pallas_call(
        _body,
        grid=(NB,),
        in_specs=[
            pl.BlockSpec((BLOCK, D), lambda i: (i, 0)),
            pl.BlockSpec((1, BLOCK, 1), lambda i: (i, 0, 0)),
            pl.BlockSpec((D, D), lambda i: (0, 0)),
            pl.BlockSpec((1, D), lambda i: (0, 0)),
            pl.BlockSpec((D, 1), lambda i: (0, 0)),
        ],
        out_specs=pl.BlockSpec((G, D), lambda i: (0, 0)),
        out_shape=jax.ShapeDtypeStruct((G, D), jnp.float32),
        scratch_shapes=[
            pltpu.VMEM((1, G), jnp.float32),
            pltpu.VMEM((1, G), jnp.float32),
            pltpu.VMEM((D, G), jnp.float32),
        ],
    )(x, ids, W, b2, q2)
